# baseline (speedup 1.0000x reference)
import jax
import jax.numpy as jnp
from jax import lax
from jax.experimental import pallas as pl
from jax.experimental.pallas import tpu as pltpu
from jax.experimental.pallas import tpu_sc as plsc

B, N, S, D = 8, 512, 256, 64
NC, NS, L = 2, 16, 16
NW = NC * NS
ROWS_PER_W = D // NW
HP = N // 2

_MESH = plsc.VectorSubcoreMesh(
    core_axis_name="c", subcore_axis_name="s",
    num_cores=NC, num_subcores=NS,
)


def _sc_lookup_body(time_hbm, space_hbm, tt_hbm, st_hbm,
                    tchunk, schunk, rowbuf_t, rowbuf_s):
    wid = lax.axis_index("s") * NC + lax.axis_index("c")
    pltpu.sync_copy(time_hbm.at[pl.ds(0, S)], tchunk)
    pltpu.sync_copy(space_hbm, schunk)
    lane = lax.iota(jnp.int32, 16)
    for r in range(ROWS_PER_W):
        d = wid * ROWS_PER_W + r
        dvec = jnp.full((L,), d, jnp.int32)
        for i in range(S // L):
            rowbuf_t[pl.ds(i * L, L)] = plsc.load_gather(tchunk, [lane + i * L, dvec])
        pltpu.sync_copy(rowbuf_t, tt_hbm.at[d])
        for i in range(N // L):
            rowbuf_s[pl.ds(i * L, L)] = plsc.load_gather(schunk, [lane + i * L, dvec])
        pltpu.sync_copy(rowbuf_s, st_hbm.at[d])


_sc_lookup = pl.kernel(
    _sc_lookup_body,
    out_type=(
        jax.ShapeDtypeStruct((D, S), jnp.float32),
        jax.ShapeDtypeStruct((D, N), jnp.float32),
    ),
    mesh=_MESH,
    scratch_types=[
        pltpu.VMEM((S, D), jnp.float32),
        pltpu.VMEM((N, D), jnp.float32),
        pltpu.VMEM((S,), jnp.float32),
        pltpu.VMEM((N,), jnp.float32),
    ],
    compiler_params=pltpu.CompilerParams(needs_layout_passes=False),
)


def _sc_planes_body(tt_hbm, st_hbm, out_hbm, colt, cols, hplane, sem):
    wid = lax.axis_index("s") * NC + lax.axis_index("c")
    for r in range(ROWS_PER_W):
        d = wid * ROWS_PER_W + r
        pltpu.sync_copy(tt_hbm.at[d], colt)
        pltpu.sync_copy(st_hbm.at[d], cols)
        for h in range(2):
            def row_body(n, carry, h=h):
                splat = plsc.load_gather(cols, [jnp.full((L,), h * HP, jnp.int32) + n])
                for i in range(S // L):
                    hplane[n, pl.ds(i * L, L)] = colt[pl.ds(i * L, L)] + splat
                return carry
            lax.fori_loop(0, HP, row_body, 0)
            copies = [
                pltpu.async_copy(hplane, out_hbm.at[b, d, pl.ds(h * HP, HP)], sem)
                for b in range(B)
            ]
            for c in copies:
                c.wait()


_sc_planes = pl.kernel(
    _sc_planes_body,
    out_type=jax.ShapeDtypeStruct((B, D, N, S), jnp.float32),
    mesh=_MESH,
    scratch_types=[
        pltpu.VMEM((S,), jnp.float32),
        pltpu.VMEM((N,), jnp.float32),
        pltpu.VMEM((HP, S), jnp.float32),
        pltpu.SemaphoreType.DMA,
    ],
    compiler_params=pltpu.CompilerParams(needs_layout_passes=False),
)


def kernel(input_ids, time_table, space_table):
    del input_ids
    tt, st = _sc_lookup(time_table, space_table)
    return _sc_planes(tt, st)


# speedup vs baseline: 2.3476x; 2.3476x over previous
import jax
import jax.numpy as jnp
from jax.experimental import pallas as pl
from jax.experimental.pallas import tpu as pltpu

B, N, S, D = 8, 512, 256, 64
DB = 16
NBUF = 3
NJ = D // DB
NSTEP = B * NJ


def _tc_body(tt_ref, st_ref, out_ref, bufs, sems):
    k = pl.program_id(0)
    b = k // NJ
    j = k % NJ
    slot = jax.lax.rem(k, NBUF)

    def _copy(kk, sl):
        bb = kk // NJ
        jj = kk % NJ
        return pltpu.make_async_copy(
            bufs.at[sl],
            out_ref.at[bb, pl.ds(jj * DB, DB)],
            sems.at[sl],
        )

    @pl.when(k >= NBUF)
    def _():
        _copy(k - NBUF, slot).wait()

    tt = tt_ref[...]
    st = st_ref[...]
    bufs[slot] = st[:, :, None] + tt[:, None, :]
    _copy(k, slot).start()

    @pl.when(k == NSTEP - 1)
    def _():
        for back in range(NBUF - 1, -1, -1):
            kk = NSTEP - 1 - back
            _copy(kk, jax.lax.rem(kk, NBUF)).wait()


def kernel(input_ids, time_table, space_table):
    del input_ids
    tt = time_table[:S].T
    st = space_table.T
    return pl.pallas_call(
        _tc_body,
        grid=(NSTEP,),
        in_specs=[
            pl.BlockSpec((DB, S), lambda k: (k % NJ, 0)),
            pl.BlockSpec((DB, N), lambda k: (k % NJ, 0)),
        ],
        out_specs=pl.BlockSpec(memory_space=pl.ANY),
        out_shape=jax.ShapeDtypeStruct((B, D, N, S), jnp.float32),
        scratch_shapes=[
            pltpu.VMEM((NBUF, DB, N, S), jnp.float32),
            pltpu.SemaphoreType.DMA((NBUF,)),
        ],
    )(tt, st)


# speedup vs baseline: 2.3645x; 1.0072x over previous
import jax
import jax.numpy as jnp
from jax.experimental import pallas as pl

B, N, S, D = 8, 512, 256, 64
DB = 16


def _tc_body(tt_ref, st_ref, out_ref):
    tt = tt_ref[...]
    st = st_ref[...]
    out_ref[0] = st[:, :, None] + tt[:, None, :]


def kernel(input_ids, time_table, space_table):
    del input_ids
    tt = time_table[:S].T
    st = space_table.T
    return pl.pallas_call(
        _tc_body,
        grid=(B, D // DB),
        in_specs=[
            pl.BlockSpec((DB, S), lambda b, j: (j, 0)),
            pl.BlockSpec((DB, N), lambda b, j: (j, 0)),
        ],
        out_specs=pl.BlockSpec((1, DB, N, S), lambda b, j: (b, j, 0, 0)),
        out_shape=jax.ShapeDtypeStruct((B, D, N, S), jnp.float32),
    )(tt, st)
